# Initial kernel scaffold; baseline (speedup 1.0000x reference)
import functools

import jax
import jax.numpy as jnp
from jax import lax
from jax.experimental import pallas as pl
from jax.experimental.pallas import tpu as pltpu
from jax.experimental.pallas import tpu_sc as plsc

_NUM_HEADS = 16
_NUM_BUCKETS = 32
_MAX_DISTANCE = 50000
_S = 2048
_NC = 2
_NS = 16
_NW = _NC * _NS
_ROWS_PER_W = _S // _NW
_LANES = 16
_VECS_PER_ROW = _S // _LANES
_NCELLS = 128
_CELL_BIAS = 1016


def _bucket_jnp(d):
    f = d.astype(jnp.float32)
    nz = f >= 1.0
    safe = jnp.where(nz, f, jnp.float32(1.0))
    lr = jnp.log(safe) / jnp.log(jnp.float32(_MAX_DISTANCE))
    lb = (lr * (_NUM_BUCKETS - 2)).astype(jnp.int32) + 1
    lb = jnp.clip(lb, 1, _NUM_BUCKETS - 1)
    return jnp.where(nz, lb, 0)


def _build_lut():
    cells = jnp.arange(_NCELLS, dtype=jnp.int32)
    dstart = jnp.ceil(
        lax.bitcast_convert_type((cells + _CELL_BIAS) << 20, jnp.float32)
    ).astype(jnp.int32)
    dend = jnp.concatenate([dstart[1:], jnp.array([65536], dtype=jnp.int32)])
    b0 = _bucket_jnp(dstart)
    delta = _bucket_jnp(dend - 1) - b0
    lo, hi = dstart, dend
    for _ in range(13):
        cont = lo < hi
        mid = (lo + hi) >> 1
        pred = _bucket_jnp(mid) > b0
        hi = jnp.where(cont & pred, mid, hi)
        lo = jnp.where(cont & jnp.logical_not(pred), mid + 1, lo)
    entry = lo | (b0 << 17) | (delta << 22)
    return jnp.where(cells == 0, 1 | (1 << 22), entry)


def _sc_body(dist_hbm, lut_hbm, tbl_hbm, out_hbm, lut_v, tbl_v, din_v, dout_v):
    wid = lax.axis_index("s") * _NC + lax.axis_index("c")
    row0 = wid * _ROWS_PER_W
    pltpu.sync_copy(lut_hbm, lut_v)
    pltpu.sync_copy(tbl_hbm, tbl_v)

    def row_body(r, carry):
        row = row0 + r
        pltpu.sync_copy(dist_hbm.at[row], din_v.at[0])

        def vec_body(j, c2):
            off = j * _LANES
            d = din_v[0, pl.ds(off, _LANES)]
            bits = plsc.bitcast(d.astype(jnp.float32), jnp.int32)
            cell = jnp.clip((bits >> 20) - _CELL_BIAS, 0, _NCELLS - 1)
            e = plsc.load_gather(lut_v, [cell])
            thr = e & 0x1FFFF
            base = (e >> 17) & 31
            delta = e >> 22
            b = base + jnp.where(d >= thr, delta, jnp.zeros_like(delta))
            for h in range(_NUM_HEADS):
                v = plsc.load_gather(tbl_v.at[h], [b])
                dout_v[0, h, pl.ds(off, _LANES)] = v
            return c2

        lax.fori_loop(0, _VECS_PER_ROW, vec_body, 0)
        for h in range(_NUM_HEADS):
            pltpu.sync_copy(dout_v.at[0, h], out_hbm.at[h, row])
        return carry

    lax.fori_loop(0, _ROWS_PER_W, row_body, 0)


@jax.jit
def _sc_call(dist, lut, tbl_t):
    fn = pl.kernel(
        _sc_body,
        out_type=jax.ShapeDtypeStruct((_NUM_HEADS, _S, _S), jnp.float32),
        mesh=plsc.VectorSubcoreMesh(
            core_axis_name="c", subcore_axis_name="s",
            num_cores=_NC, num_subcores=_NS,
        ),
        scratch_types=[
            pltpu.VMEM((_NCELLS,), jnp.int32),
            pltpu.VMEM((_NUM_HEADS, _NUM_BUCKETS), jnp.float32),
            pltpu.VMEM((2, _S), jnp.int32),
            pltpu.VMEM((2, _NUM_HEADS, _S), jnp.float32),
        ],
    )
    return fn(dist, lut, tbl_t)


def kernel(distances, table):
    dist = distances.reshape(_S, _S)
    out = _sc_call(dist, _build_lut(), table.T)
    return out[None]


# speedup vs baseline: 18.2627x; 18.2627x over previous
import functools

import jax
import jax.numpy as jnp
from jax import lax
from jax.experimental import pallas as pl
from jax.experimental.pallas import tpu as pltpu
from jax.experimental.pallas import tpu_sc as plsc

_NUM_HEADS = 16
_NUM_BUCKETS = 32
_MAX_DISTANCE = 50000
_S = 2048
_NC = 2
_NS = 16
_NW = _NC * _NS
_ROWS_PER_W = _S // _NW
_LANES = 16
_VECS_PER_ROW = _S // _LANES
_NCELLS = 128
_CELL_BIAS = 1016


def _bucket_jnp(d):
    f = d.astype(jnp.float32)
    nz = f >= 1.0
    safe = jnp.where(nz, f, jnp.float32(1.0))
    lr = jnp.log(safe) / jnp.log(jnp.float32(_MAX_DISTANCE))
    lb = (lr * (_NUM_BUCKETS - 2)).astype(jnp.int32) + 1
    lb = jnp.clip(lb, 1, _NUM_BUCKETS - 1)
    return jnp.where(nz, lb, 0)


def _build_lut():
    cells = jnp.arange(_NCELLS, dtype=jnp.int32)
    dstart = jnp.ceil(
        lax.bitcast_convert_type((cells + _CELL_BIAS) << 20, jnp.float32)
    ).astype(jnp.int32)
    dend = jnp.concatenate([dstart[1:], jnp.array([65536], dtype=jnp.int32)])
    b0 = _bucket_jnp(dstart)
    delta = _bucket_jnp(dend - 1) - b0
    lo, hi = dstart, dend
    for _ in range(13):
        cont = lo < hi
        mid = (lo + hi) >> 1
        pred = _bucket_jnp(mid) > b0
        hi = jnp.where(cont & pred, mid, hi)
        lo = jnp.where(cont & jnp.logical_not(pred), mid + 1, lo)
    entry = lo | (b0 << 17) | (delta << 22)
    return jnp.where(cells == 0, 1 | (1 << 22), entry)


def _sc_body(dist_hbm, lut_hbm, tbl_hbm, out_hbm, lut_v, tbl_v, din_v, dout_v):
    wid = lax.axis_index("s") * _NC + lax.axis_index("c")
    row0 = wid * _ROWS_PER_W
    pltpu.sync_copy(lut_hbm, lut_v)
    pltpu.sync_copy(tbl_hbm, tbl_v)

    def row_body(r, carry):
        row = row0 + r
        pltpu.sync_copy(dist_hbm.at[row], din_v.at[0])

        def vec_body(j, c2):
            off = j * _LANES
            d = din_v[0, pl.ds(off, _LANES)]
            bits = lax.bitcast_convert_type(d.astype(jnp.float32), jnp.int32)
            cell = jnp.clip((bits >> 20) - _CELL_BIAS, 0, _NCELLS - 1)
            e = plsc.load_gather(lut_v, [cell])
            thr = e & 0x1FFFF
            base = (e >> 17) & 31
            delta = e >> 22
            b = base + jnp.where(d >= thr, delta, jnp.zeros_like(delta))
            for h in range(_NUM_HEADS):
                v = plsc.load_gather(tbl_v.at[h], [b])
                dout_v[0, h, pl.ds(off, _LANES)] = v
            return c2

        lax.fori_loop(0, _VECS_PER_ROW, vec_body, 0)
        for h in range(_NUM_HEADS):
            pltpu.sync_copy(dout_v.at[0, h], out_hbm.at[h, row])
        return carry

    lax.fori_loop(0, _ROWS_PER_W, row_body, 0)


@jax.jit
def _sc_call(dist, lut, tbl_t):
    fn = pl.kernel(
        _sc_body,
        out_type=jax.ShapeDtypeStruct((_NUM_HEADS, _S, _S), jnp.float32),
        mesh=plsc.VectorSubcoreMesh(
            core_axis_name="c", subcore_axis_name="s",
            num_cores=_NC, num_subcores=_NS,
        ),
        scratch_types=[
            pltpu.VMEM((_NCELLS,), jnp.int32),
            pltpu.VMEM((_NUM_HEADS, _NUM_BUCKETS), jnp.float32),
            pltpu.VMEM((2, _S), jnp.int32),
            pltpu.VMEM((2, _NUM_HEADS, _S), jnp.float32),
        ],
        compiler_params=pltpu.CompilerParams(needs_layout_passes=False),
    )
    return fn(dist, lut, tbl_t)


def kernel(distances, table):
    dist = distances.reshape(_S, _S)
    out = _sc_call(dist, _build_lut(), table.T)
    return out[None]


# speedup vs baseline: 23.1304x; 1.2665x over previous
import functools

import jax
import jax.numpy as jnp
from jax import lax
from jax.experimental import pallas as pl
from jax.experimental.pallas import tpu as pltpu
from jax.experimental.pallas import tpu_sc as plsc

_NUM_HEADS = 16
_NUM_BUCKETS = 32
_MAX_DISTANCE = 50000
_S = 2048
_NC = 2
_NS = 16
_NW = _NC * _NS
_ROWS_PER_W = _S // _NW
_LANES = 16
_VECS_PER_ROW = _S // _LANES
_NCELLS = 128
_CELL_BIAS = 1016


def _bucket_jnp(d):
    f = d.astype(jnp.float32)
    nz = f >= 1.0
    safe = jnp.where(nz, f, jnp.float32(1.0))
    lr = jnp.log(safe) / jnp.log(jnp.float32(_MAX_DISTANCE))
    lb = (lr * (_NUM_BUCKETS - 2)).astype(jnp.int32) + 1
    lb = jnp.clip(lb, 1, _NUM_BUCKETS - 1)
    return jnp.where(nz, lb, 0)


def _build_lut():
    cells = jnp.arange(_NCELLS, dtype=jnp.int32)
    dstart = jnp.ceil(
        lax.bitcast_convert_type((cells + _CELL_BIAS) << 20, jnp.float32)
    ).astype(jnp.int32)
    dend = jnp.concatenate([dstart[1:], jnp.array([65536], dtype=jnp.int32)])
    b0 = _bucket_jnp(dstart)
    delta = _bucket_jnp(dend - 1) - b0
    lo, hi = dstart, dend
    for _ in range(13):
        cont = lo < hi
        mid = (lo + hi) >> 1
        pred = _bucket_jnp(mid) > b0
        hi = jnp.where(cont & pred, mid, hi)
        lo = jnp.where(cont & jnp.logical_not(pred), mid + 1, lo)
    entry = lo | (b0 << 17) | (delta << 22)
    return jnp.where(cells == 0, 1 | (1 << 22), entry)


def _sc_body(dist_hbm, lut_hbm, tbl_hbm, out_hbm, lut_v, tbl_v, din_v, dout_v,
             sem_in0, sem_in1, sem_out0, sem_out1):
    wid = lax.axis_index("s") * _NC + lax.axis_index("c")
    row0 = wid * _ROWS_PER_W
    sem_in = (sem_in0, sem_in1)
    sem_out = (sem_out0, sem_out1)
    pltpu.sync_copy(lut_hbm, lut_v)
    pltpu.sync_copy(tbl_hbm, tbl_v)

    def compute_row(slot):
        def vec_body(j, c2):
            off = j * _LANES
            d = din_v[slot, pl.ds(off, _LANES)]
            bits = lax.bitcast_convert_type(d.astype(jnp.float32), jnp.int32)
            cell = jnp.clip((bits >> 20) - _CELL_BIAS, 0, _NCELLS - 1)
            e = plsc.load_gather(lut_v, [cell])
            thr = e & 0x1FFFF
            base = (e >> 17) & 31
            delta = e >> 22
            b = base + jnp.where(d >= thr, delta, jnp.zeros_like(delta))
            for h in range(_NUM_HEADS):
                v = plsc.load_gather(tbl_v.at[h], [b])
                dout_v[slot, h, pl.ds(off, _LANES)] = v
            return c2

        lax.fori_loop(0, _VECS_PER_ROW, vec_body, 0, unroll=4)

    for b in range(2):
        pltpu.async_copy(dist_hbm.at[row0 + b], din_v.at[b], sem_in[b])

    def pair_body(p, carry):
        for b in range(2):
            row = row0 + p * 2 + b
            pltpu.make_async_copy(dist_hbm.at[row], din_v.at[b],
                                  sem_in[b]).wait()

            @pl.when(p > 0)
            def _drain():
                for h in range(_NUM_HEADS):
                    pltpu.make_async_copy(dout_v.at[b, h],
                                          out_hbm.at[h, row - 2],
                                          sem_out[b]).wait()

            compute_row(b)
            for h in range(_NUM_HEADS):
                pltpu.async_copy(dout_v.at[b, h], out_hbm.at[h, row],
                                 sem_out[b])

            @pl.when(p * 2 + b + 2 < _ROWS_PER_W)
            def _next_in():
                pltpu.async_copy(dist_hbm.at[row + 2], din_v.at[b], sem_in[b])

        return carry

    lax.fori_loop(0, _ROWS_PER_W // 2, pair_body, 0)

    for b in range(2):
        row = row0 + _ROWS_PER_W - 2 + b
        for h in range(_NUM_HEADS):
            pltpu.make_async_copy(dout_v.at[b, h], out_hbm.at[h, row],
                                  sem_out[b]).wait()


@jax.jit
def _sc_call(dist, lut, tbl_t):
    fn = pl.kernel(
        _sc_body,
        out_type=jax.ShapeDtypeStruct((_NUM_HEADS, _S, _S), jnp.float32),
        mesh=plsc.VectorSubcoreMesh(
            core_axis_name="c", subcore_axis_name="s",
            num_cores=_NC, num_subcores=_NS,
        ),
        scratch_types=[
            pltpu.VMEM((_NCELLS,), jnp.int32),
            pltpu.VMEM((_NUM_HEADS, _NUM_BUCKETS), jnp.float32),
            pltpu.VMEM((2, _S), jnp.int32),
            pltpu.VMEM((2, _NUM_HEADS, _S), jnp.float32),
            pltpu.SemaphoreType.DMA,
            pltpu.SemaphoreType.DMA,
            pltpu.SemaphoreType.DMA,
            pltpu.SemaphoreType.DMA,
        ],
        compiler_params=pltpu.CompilerParams(needs_layout_passes=False),
    )
    return fn(dist, lut, tbl_t)


def kernel(distances, table):
    dist = distances.reshape(_S, _S)
    out = _sc_call(dist, _build_lut(), table.T)
    return out[None]


# speedup vs baseline: 78.1275x; 3.3777x over previous
import functools

import jax
import jax.numpy as jnp
from jax import lax
from jax.experimental import pallas as pl
from jax.experimental.pallas import tpu as pltpu
from jax.experimental.pallas import tpu_sc as plsc

_NUM_HEADS = 16
_NUM_BUCKETS = 32
_MAX_DISTANCE = 50000
_S = 2048
_NC = 2
_NS = 16
_NW = _NC * _NS
_ROWS_PER_W = _S // _NW
_LANES = 16
_VECS_PER_ROW = _S // _LANES
_NCELLS = 128
_CELL_BIAS = 1016


def _bucket_jnp(d):
    f = d.astype(jnp.float32)
    nz = f >= 1.0
    safe = jnp.where(nz, f, jnp.float32(1.0))
    lr = jnp.log(safe) / jnp.log(jnp.float32(_MAX_DISTANCE))
    lb = (lr * (_NUM_BUCKETS - 2)).astype(jnp.int32) + 1
    lb = jnp.clip(lb, 1, _NUM_BUCKETS - 1)
    return jnp.where(nz, lb, 0)


def _build_lut():
    cells = jnp.arange(_NCELLS, dtype=jnp.int32)
    dstart = jnp.ceil(
        lax.bitcast_convert_type((cells + _CELL_BIAS) << 20, jnp.float32)
    ).astype(jnp.int32)
    dend = jnp.concatenate([dstart[1:], jnp.array([65536], dtype=jnp.int32)])
    b0 = _bucket_jnp(dstart)
    delta = _bucket_jnp(dend - 1) - b0
    lo, hi = dstart, dend
    for _ in range(13):
        cont = lo < hi
        mid = (lo + hi) >> 1
        pred = _bucket_jnp(mid) > b0
        hi = jnp.where(cont & pred, mid, hi)
        lo = jnp.where(cont & jnp.logical_not(pred), mid + 1, lo)
    entry = lo | (b0 << 17) | (delta << 22)
    return jnp.where(cells == 0, 1 | (1 << 22), entry)


def _sc_body(dist_hbm, lut_hbm, tbl_hbm, out_hbm, lut_v, tbl_v, din_v, dout_v,
             sem_in0, sem_in1, sem_out0, sem_out1):
    wid = lax.axis_index("s") * _NC + lax.axis_index("c")
    row0 = wid * _ROWS_PER_W
    sem_in = (sem_in0, sem_in1)
    sem_out = (sem_out0, sem_out1)
    pltpu.sync_copy(lut_hbm, lut_v)
    pltpu.sync_copy(tbl_hbm, tbl_v)

    def compute_row(slot):
        @plsc.parallel_loop(0, _VECS_PER_ROW, 1, unroll=4)
        def vec_body(j):
            off = j * _LANES
            d = din_v[slot, pl.ds(off, _LANES)]
            bits = lax.bitcast_convert_type(d.astype(jnp.float32), jnp.int32)
            cell = jnp.clip((bits >> 20) - _CELL_BIAS, 0, _NCELLS - 1)
            e = plsc.load_gather(lut_v, [cell])
            thr = e & 0x1FFFF
            base = (e >> 17) & 31
            delta = e >> 22
            b = base + jnp.where(d >= thr, delta, jnp.zeros_like(delta))
            for h in range(_NUM_HEADS):
                v = plsc.load_gather(tbl_v.at[h], [b])
                dout_v[slot, h, pl.ds(off, _LANES)] = v

    for b in range(2):
        pltpu.async_copy(dist_hbm.at[row0 + b], din_v.at[b], sem_in[b])

    def pair_body(p, carry):
        for b in range(2):
            row = row0 + p * 2 + b
            pltpu.make_async_copy(dist_hbm.at[row], din_v.at[b],
                                  sem_in[b]).wait()

            @pl.when(p > 0)
            def _drain():
                for h in range(_NUM_HEADS):
                    pltpu.make_async_copy(dout_v.at[b, h],
                                          out_hbm.at[h, row - 2],
                                          sem_out[b]).wait()

            compute_row(b)
            for h in range(_NUM_HEADS):
                pltpu.async_copy(dout_v.at[b, h], out_hbm.at[h, row],
                                 sem_out[b])

            @pl.when(p * 2 + b + 2 < _ROWS_PER_W)
            def _next_in():
                pltpu.async_copy(dist_hbm.at[row + 2], din_v.at[b], sem_in[b])

        return carry

    lax.fori_loop(0, _ROWS_PER_W // 2, pair_body, 0)

    for b in range(2):
        row = row0 + _ROWS_PER_W - 2 + b
        for h in range(_NUM_HEADS):
            pltpu.make_async_copy(dout_v.at[b, h], out_hbm.at[h, row],
                                  sem_out[b]).wait()


@jax.jit
def _sc_call(dist, lut, tbl_t):
    fn = pl.kernel(
        _sc_body,
        out_type=jax.ShapeDtypeStruct((_NUM_HEADS, _S, _S), jnp.float32),
        mesh=plsc.VectorSubcoreMesh(
            core_axis_name="c", subcore_axis_name="s",
            num_cores=_NC, num_subcores=_NS,
        ),
        scratch_types=[
            pltpu.VMEM((_NCELLS,), jnp.int32),
            pltpu.VMEM((_NUM_HEADS, _NUM_BUCKETS), jnp.float32),
            pltpu.VMEM((2, _S), jnp.int32),
            pltpu.VMEM((2, _NUM_HEADS, _S), jnp.float32),
            pltpu.SemaphoreType.DMA,
            pltpu.SemaphoreType.DMA,
            pltpu.SemaphoreType.DMA,
            pltpu.SemaphoreType.DMA,
        ],
        compiler_params=pltpu.CompilerParams(needs_layout_passes=False),
    )
    return fn(dist, lut, tbl_t)


def kernel(distances, table):
    dist = distances.reshape(_S, _S)
    out = _sc_call(dist, _build_lut(), table.T)
    return out[None]


# speedup vs baseline: 80.1810x; 1.0263x over previous
import functools

import jax
import jax.numpy as jnp
from jax import lax
from jax.experimental import pallas as pl
from jax.experimental.pallas import tpu as pltpu
from jax.experimental.pallas import tpu_sc as plsc

_NUM_HEADS = 16
_NUM_BUCKETS = 32
_MAX_DISTANCE = 50000
_S = 2048
_NC = 2
_NS = 16
_NW = _NC * _NS
_ROWS_PER_W = _S // _NW
_LANES = 16
_VECS_PER_ROW = _S // _LANES
_NCELLS = 128
_CELL_BIAS = 1016


def _bucket_jnp(d):
    f = d.astype(jnp.float32)
    nz = f >= 1.0
    safe = jnp.where(nz, f, jnp.float32(1.0))
    lr = jnp.log(safe) / jnp.log(jnp.float32(_MAX_DISTANCE))
    lb = (lr * (_NUM_BUCKETS - 2)).astype(jnp.int32) + 1
    lb = jnp.clip(lb, 1, _NUM_BUCKETS - 1)
    return jnp.where(nz, lb, 0)


def _build_lut():
    cells = jnp.arange(_NCELLS, dtype=jnp.int32)
    dstart = jnp.ceil(
        lax.bitcast_convert_type((cells + _CELL_BIAS) << 20, jnp.float32)
    ).astype(jnp.int32)
    dend = jnp.concatenate([dstart[1:], jnp.array([65536], dtype=jnp.int32)])
    b0 = _bucket_jnp(dstart)
    delta = _bucket_jnp(dend - 1) - b0
    lo, hi = dstart, dend
    for _ in range(13):
        cont = lo < hi
        mid = (lo + hi) >> 1
        pred = _bucket_jnp(mid) > b0
        hi = jnp.where(cont & pred, mid, hi)
        lo = jnp.where(cont & jnp.logical_not(pred), mid + 1, lo)
    entry = lo | (b0 << 17) | (delta << 22)
    return jnp.where(cells == 0, 1 | (1 << 22), entry)


def _sc_body(dist_hbm, lut_hbm, tbl_hbm, out_hbm, lut_v, tbl_v, din_v, dout_v,
             sem_in0, sem_in1, sem_out0, sem_out1):
    wid = lax.axis_index("s") * _NC + lax.axis_index("c")
    row0 = wid * _ROWS_PER_W
    sem_in = (sem_in0, sem_in1)
    sem_out = (sem_out0, sem_out1)
    pltpu.sync_copy(lut_hbm, lut_v)
    pltpu.sync_copy(tbl_hbm, tbl_v)

    def compute_row(slot):
        @plsc.parallel_loop(0, _VECS_PER_ROW, 1, unroll=8)
        def vec_body(j):
            off = j * _LANES
            d = din_v[slot, pl.ds(off, _LANES)]
            bits = lax.bitcast_convert_type(d.astype(jnp.float32), jnp.int32)
            cell = jnp.clip((bits >> 20) - _CELL_BIAS, 0, _NCELLS - 1)
            e = plsc.load_gather(lut_v, [cell])
            thr = e & 0x1FFFF
            base = (e >> 17) & 31
            delta = e >> 22
            b = base + jnp.where(d >= thr, delta, jnp.zeros_like(delta))
            for h in range(_NUM_HEADS):
                v = plsc.load_gather(tbl_v.at[h], [b])
                dout_v[slot, h, pl.ds(off, _LANES)] = v

    for b in range(2):
        pltpu.async_copy(dist_hbm.at[row0 + b], din_v.at[b], sem_in[b])

    def pair_body(p, carry):
        for b in range(2):
            row = row0 + p * 2 + b
            pltpu.make_async_copy(dist_hbm.at[row], din_v.at[b],
                                  sem_in[b]).wait()

            @pl.when(p > 0)
            def _drain():
                for h in range(_NUM_HEADS):
                    pltpu.make_async_copy(dout_v.at[b, h],
                                          out_hbm.at[h, row - 2],
                                          sem_out[b]).wait()

            compute_row(b)
            for h in range(_NUM_HEADS):
                pltpu.async_copy(dout_v.at[b, h], out_hbm.at[h, row],
                                 sem_out[b])

            @pl.when(p * 2 + b + 2 < _ROWS_PER_W)
            def _next_in():
                pltpu.async_copy(dist_hbm.at[row + 2], din_v.at[b], sem_in[b])

        return carry

    lax.fori_loop(0, _ROWS_PER_W // 2, pair_body, 0)

    for b in range(2):
        row = row0 + _ROWS_PER_W - 2 + b
        for h in range(_NUM_HEADS):
            pltpu.make_async_copy(dout_v.at[b, h], out_hbm.at[h, row],
                                  sem_out[b]).wait()


@jax.jit
def _sc_call(dist, lut, tbl_t):
    fn = pl.kernel(
        _sc_body,
        out_type=jax.ShapeDtypeStruct((_NUM_HEADS, _S, _S), jnp.float32),
        mesh=plsc.VectorSubcoreMesh(
            core_axis_name="c", subcore_axis_name="s",
            num_cores=_NC, num_subcores=_NS,
        ),
        scratch_types=[
            pltpu.VMEM((_NCELLS,), jnp.int32),
            pltpu.VMEM((_NUM_HEADS, _NUM_BUCKETS), jnp.float32),
            pltpu.VMEM((2, _S), jnp.int32),
            pltpu.VMEM((2, _NUM_HEADS, _S), jnp.float32),
            pltpu.SemaphoreType.DMA,
            pltpu.SemaphoreType.DMA,
            pltpu.SemaphoreType.DMA,
            pltpu.SemaphoreType.DMA,
        ],
        compiler_params=pltpu.CompilerParams(needs_layout_passes=False),
    )
    return fn(dist, lut, tbl_t)


def kernel(distances, table):
    dist = distances.reshape(_S, _S)
    out = _sc_call(dist, _build_lut(), table.T)
    return out[None]


# speedup vs baseline: 103.8647x; 1.2954x over previous
import functools

import jax
import jax.numpy as jnp
from jax import lax
from jax.experimental import pallas as pl
from jax.experimental.pallas import tpu as pltpu
from jax.experimental.pallas import tpu_sc as plsc

_NUM_HEADS = 16
_NUM_BUCKETS = 32
_MAX_DISTANCE = 50000
_S = 2048
_NC = 2
_NS = 16
_NW = _NC * _NS
_ROWS_PER_W = _S // _NW
_LANES = 16
_VECS_PER_ROW = _S // _LANES
_NCELLS = 128
_CELL_BIAS = 1016


def _bucket_jnp(d):
    f = d.astype(jnp.float32)
    nz = f >= 1.0
    safe = jnp.where(nz, f, jnp.float32(1.0))
    lr = jnp.log(safe) / jnp.log(jnp.float32(_MAX_DISTANCE))
    lb = (lr * (_NUM_BUCKETS - 2)).astype(jnp.int32) + 1
    lb = jnp.clip(lb, 1, _NUM_BUCKETS - 1)
    return jnp.where(nz, lb, 0)


def _build_lut():
    cells = jnp.arange(_NCELLS, dtype=jnp.int32)
    dstart = jnp.ceil(
        lax.bitcast_convert_type((cells + _CELL_BIAS) << 20, jnp.float32)
    ).astype(jnp.int32)
    dend = jnp.concatenate([dstart[1:], jnp.array([65536], dtype=jnp.int32)])
    b0 = _bucket_jnp(dstart)
    delta = _bucket_jnp(dend - 1) - b0
    lo, hi = dstart, dend
    for _ in range(13):
        cont = lo < hi
        mid = (lo + hi) >> 1
        pred = _bucket_jnp(mid) > b0
        hi = jnp.where(cont & pred, mid, hi)
        lo = jnp.where(cont & jnp.logical_not(pred), mid + 1, lo)
    entry = lo | (b0 << 17) | (delta << 22)
    return jnp.where(cells == 0, 1 | (1 << 22), entry)


def _sc_body(dist_hbm, lut_hbm, tbl_hbm, out_hbm, lut_v, tbl_v, din_v, dout_v,
             sem_in0, sem_in1, sem_out0, sem_out1):
    wid = lax.axis_index("s") * _NC + lax.axis_index("c")
    row0 = wid * _ROWS_PER_W
    sem_in = (sem_in0, sem_in1)
    sem_out = (sem_out0, sem_out1)
    pltpu.sync_copy(lut_hbm, lut_v)
    pltpu.sync_copy(tbl_hbm, tbl_v)

    def compute_row(slot):
        @plsc.parallel_loop(0, _VECS_PER_ROW, 1, unroll=8)
        def vec_body(j):
            off = j * _LANES
            d = din_v[slot, pl.ds(off, _LANES)]
            bits = lax.bitcast_convert_type(d.astype(jnp.float32), jnp.int32)
            cell = jnp.clip((bits >> 20) - _CELL_BIAS, 0, _NCELLS - 1)
            e = plsc.load_gather(lut_v, [cell])
            thr = e & 0x1FFFF
            base = (e >> 17) & 31
            delta = e >> 22
            b = base + jnp.where(d >= thr, delta, jnp.zeros_like(delta))
            for h in range(_NUM_HEADS):
                v = plsc.load_gather(tbl_v.at[h], [b])
                dout_v[slot, h, pl.ds(off, _LANES)] = v

    for b in range(2):
        pltpu.async_copy(dist_hbm.at[row0 + b], din_v.at[b], sem_in[b])

    def pair_body(p, carry):
        for b in range(2):
            row = row0 + p * 2 + b
            pltpu.make_async_copy(dist_hbm.at[row], din_v.at[b],
                                  sem_in[b]).wait()

            @pl.when(p > 0)
            def _drain():
                pltpu.make_async_copy(dout_v.at[b], out_hbm.at[:, row - 2],
                                      sem_out[b]).wait()

            compute_row(b)
            pltpu.async_copy(dout_v.at[b], out_hbm.at[:, row], sem_out[b])

            @pl.when(p * 2 + b + 2 < _ROWS_PER_W)
            def _next_in():
                pltpu.async_copy(dist_hbm.at[row + 2], din_v.at[b], sem_in[b])

        return carry

    lax.fori_loop(0, _ROWS_PER_W // 2, pair_body, 0)

    for b in range(2):
        row = row0 + _ROWS_PER_W - 2 + b
        pltpu.make_async_copy(dout_v.at[b], out_hbm.at[:, row],
                              sem_out[b]).wait()


@jax.jit
def _sc_call(dist, lut, tbl_t):
    fn = pl.kernel(
        _sc_body,
        out_type=jax.ShapeDtypeStruct((_NUM_HEADS, _S, _S), jnp.float32),
        mesh=plsc.VectorSubcoreMesh(
            core_axis_name="c", subcore_axis_name="s",
            num_cores=_NC, num_subcores=_NS,
        ),
        scratch_types=[
            pltpu.VMEM((_NCELLS,), jnp.int32),
            pltpu.VMEM((_NUM_HEADS, _NUM_BUCKETS), jnp.float32),
            pltpu.VMEM((2, _S), jnp.int32),
            pltpu.VMEM((2, _NUM_HEADS, _S), jnp.float32),
            pltpu.SemaphoreType.DMA,
            pltpu.SemaphoreType.DMA,
            pltpu.SemaphoreType.DMA,
            pltpu.SemaphoreType.DMA,
        ],
        compiler_params=pltpu.CompilerParams(needs_layout_passes=False),
    )
    return fn(dist, lut, tbl_t)


def kernel(distances, table):
    dist = distances.reshape(_S, _S)
    out = _sc_call(dist, _build_lut(), table.T)
    return out[None]
